# Initial kernel scaffold; baseline (speedup 1.0000x reference)
import jax
import jax.numpy as jnp
from jax.experimental import pallas as pl


def kernel(x, edge_index_rel0, edge_index_rel1, edge_index_rel2, W1, al1, ar1, b1, W2, al2, ar2, b2, W3, al3, ar3, b3):
    raise NotImplementedError("write your pallas kernel here")


# speedup vs baseline: 13.2509x; 13.2509x over previous
import functools

import jax
import jax.numpy as jnp
from jax.experimental import pallas as pl

_N = 50000
_R = 3
_H = 2
_DOUT = 64
_BN = 400
_BC = 400


def _proj_body(feat_ref, w_ref, al_ref, ar_ref, h_ref, el_ref, er_ref):
    f = feat_ref[...]
    w = w_ref[0]
    h = jnp.dot(f, w, preferred_element_type=jnp.float32)
    h_ref[0] = h
    hr = h.reshape(_BN, _H, _DOUT)
    el_ref[0] = jnp.sum(hr * al_ref[0][None], axis=-1)
    er_ref[0] = jnp.sum(hr * ar_ref[0][None], axis=-1)


def _proj(feat, w, al, ar):
    rows, din = feat.shape
    nb = rows // _BN
    hd = _H * _DOUT
    return pl.pallas_call(
        _proj_body,
        grid=(_R, nb),
        in_specs=[
            pl.BlockSpec((_BN, din), lambda r, j: (j, 0)),
            pl.BlockSpec((1, din, hd), lambda r, j: (r, 0, 0)),
            pl.BlockSpec((1, _H, _DOUT), lambda r, j: (r, 0, 0)),
            pl.BlockSpec((1, _H, _DOUT), lambda r, j: (r, 0, 0)),
        ],
        out_specs=[
            pl.BlockSpec((1, _BN, hd), lambda r, j: (r, j, 0)),
            pl.BlockSpec((1, _BN, _H), lambda r, j: (r, j, 0)),
            pl.BlockSpec((1, _BN, _H), lambda r, j: (r, j, 0)),
        ],
        out_shape=[
            jax.ShapeDtypeStruct((_R, rows, hd), jnp.float32),
            jax.ShapeDtypeStruct((_R, rows, _H), jnp.float32),
            jax.ShapeDtypeStruct((_R, rows, _H), jnp.float32),
        ],
    )(feat, w, al, ar)


def _combine_body(s_ref, den_ref, b_ref, out_ref, *, ph, relu, mean):
    s = s_ref[...]
    den = den_ref[...]
    sden = jnp.where(den == 0.0, 1.0, den)
    rst = s.reshape(_R, _BC, ph, _DOUT) / sden[..., None]
    tot = jnp.sum(rst, axis=0)
    p = ph // _H
    tot = tot.reshape(_BC, p, _H, _DOUT) + b_ref[...][None, None]
    if mean:
        out_ref[...] = jnp.mean(tot, axis=(1, 2))
    else:
        tot = tot.reshape(_BC, ph * _DOUT)
        out_ref[...] = jnp.maximum(tot, 0.0) if relu else tot


def _combine(s, den, bsum, ph, relu, mean):
    nb = _N // _BC
    f = ph * _DOUT
    out_cols = _DOUT if mean else f
    body = functools.partial(_combine_body, ph=ph, relu=relu, mean=mean)
    return pl.pallas_call(
        body,
        grid=(nb,),
        in_specs=[
            pl.BlockSpec((_R, _BC, f), lambda j: (0, j, 0)),
            pl.BlockSpec((_R, _BC, ph), lambda j: (0, j, 0)),
            pl.BlockSpec((_H, _DOUT), lambda j: (0, 0)),
        ],
        out_specs=pl.BlockSpec((_BC, out_cols), lambda j: (j, 0)),
        out_shape=jax.ShapeDtypeStruct((_N, out_cols), jnp.float32),
    )(s, den, bsum)


def _edge_aggregate(h_r, el_r, er_r, src, dst, ph):
    e = el_r[src] + er_r[dst]
    e = jnp.where(e > 0, e, 0.2 * e)
    emax = jax.ops.segment_max(e, dst, num_segments=_N)
    ee = jnp.exp(e - emax[dst])
    den = jax.ops.segment_sum(ee, dst, num_segments=_N)
    hs = h_r[src].reshape(-1, ph, _DOUT) * ee[..., None]
    s = jax.ops.segment_sum(hs.reshape(-1, ph * _DOUT), dst, num_segments=_N)
    return s, den


def _layer(feat, edges, w, al, ar, b, p_in, relu, mean):
    h, el, er = _proj(feat, w, al, ar)
    ph = p_in * _H
    hv = h.reshape(_R, _N, ph * _DOUT)
    elv = el.reshape(_R, _N, ph)
    erv = er.reshape(_R, _N, ph)
    ss, dd = [], []
    for r in range(_R):
        s_r, den_r = _edge_aggregate(hv[r], elv[r], erv[r], edges[r][0], edges[r][1], ph)
        ss.append(s_r)
        dd.append(den_r)
    s = jnp.stack(ss)
    den = jnp.stack(dd)
    return _combine(s, den, jnp.sum(b, axis=0), ph, relu, mean)


def kernel(x, edge_index_rel0, edge_index_rel1, edge_index_rel2,
           W1, al1, ar1, b1, W2, al2, ar2, b2, W3, al3, ar3, b3):
    edges = [edge_index_rel0, edge_index_rel1, edge_index_rel2]
    f = _layer(x, edges, W1, al1, ar1, b1, 1, True, False)
    f = _layer(f.reshape(_N * 2, _DOUT), edges, W2, al2, ar2, b2, 2, True, False)
    f = _layer(f.reshape(_N * 4, _DOUT), edges, W3, al3, ar3, b3, 4, False, True)
    return f
